# baseline (speedup 1.0000x reference)
import functools

import jax
import jax.numpy as jnp
from jax import lax
from jax.experimental import pallas as pl
from jax.experimental.pallas import tpu as pltpu
from jax.experimental.pallas import tpu_sc as plsc

EMBED = 64
NC = 2
NS = 16
NW = NC * NS
BLK = 128


@functools.lru_cache(maxsize=None)
def _build(B, T):
    mesh = plsc.VectorSubcoreMesh(core_axis_name="c", subcore_axis_name="s")

    @functools.partial(
        pl.kernel,
        out_type=jax.ShapeDtypeStruct((T, 8, B // BLK, 8, BLK), jnp.float32),
        mesh=mesh,
        compiler_params=pltpu.CompilerParams(
            use_tc_tiling_on_sc=False, needs_layout_passes=False
        ),
        scratch_types=[
            pltpu.VMEM((T, BLK), jnp.int32),
            pltpu.VMEM((BLK, EMBED), jnp.float32),
            pltpu.VMEM((BLK, EMBED), jnp.float32),
            pltpu.VMEM((8, 8, BLK), jnp.float32),
            pltpu.VMEM((8, 8, BLK), jnp.float32),
            pltpu.SemaphoreType.DMA,
            pltpu.SemaphoreType.DMA,
            pltpu.SemaphoreType.DMA,
            pltpu.SemaphoreType.DMA,
        ],
    )
    def emb(xt_hbm, table_hbm, out_hbm, idx_v, rows0, rows1, t0, t1,
            gsem0, gsem1, osem0, osem1):
        wid = lax.axis_index("s") * NC + lax.axis_index("c")
        pltpu.sync_copy(xt_hbm.at[:, wid], idx_v)

        bufs = (rows0, rows1)
        tbufs = (t0, t1)
        gsems = (gsem0, gsem1)
        osems = (osem0, osem1)

        row_idx = [lax.iota(jnp.int32, 16) + g * 16 for g in range(8)]

        def fire_gather(t, b):
            pltpu.async_copy(
                table_hbm.at[idx_v.at[t]], bufs[b], gsems[b]
            )

        def drain_gather(b):
            pltpu.make_async_copy(
                table_hbm.at[pl.ds(0, BLK)], bufs[b], gsems[b]
            ).wait()

        def drain_out(b):
            pltpu.make_async_copy(
                out_hbm.at[0, :, 0], tbufs[b], osems[b]
            ).wait()

        def transpose(b):
            for e in range(EMBED):
                e8, er = divmod(e, 8)
                col = jnp.full((16,), e, jnp.int32)
                for g in range(8):
                    v = plsc.load_gather(bufs[b], [row_idx[g], col])
                    tbufs[b][e8, er, pl.ds(g * 16, 16)] = v

        fire_gather(0, 0)
        fire_gather(1, 1)

        @pl.loop(0, T // 2)
        def _(p):
            for b in range(2):
                t = p * 2 + b
                drain_gather(b)

                @pl.when(t >= 2)
                def _():
                    drain_out(b)

                transpose(b)
                pltpu.async_copy(tbufs[b], out_hbm.at[t, :, wid], osems[b])

                @pl.when(t + 2 < T)
                def _():
                    fire_gather(t + 2, b)

        drain_out(0)
        drain_out(1)

    return emb


def kernel(x, table):
    B, T = x.shape
    xt = x.astype(jnp.int32).T.reshape(T, NW, BLK)
    out6 = _build(B, T)(xt, table)
    return out6.transpose(2, 4, 0, 1, 3).reshape(B, T, EMBED)


# speedup vs baseline: 2.0585x; 2.0585x over previous
import functools

import jax
import jax.numpy as jnp
from jax import lax
from jax.experimental import pallas as pl
from jax.experimental.pallas import tpu as pltpu
from jax.experimental.pallas import tpu_sc as plsc

EMBED = 64
NC = 2
NS = 16
NW = NC * NS
BLK = 128


@functools.lru_cache(maxsize=None)
def _build(B, T):
    mesh = plsc.VectorSubcoreMesh(core_axis_name="c", subcore_axis_name="s")

    @functools.partial(
        pl.kernel,
        out_type=jax.ShapeDtypeStruct((T, 8, B // BLK, 8, BLK), jnp.float32),
        mesh=mesh,
        compiler_params=pltpu.CompilerParams(
            use_tc_tiling_on_sc=False, needs_layout_passes=False
        ),
        scratch_types=[
            pltpu.VMEM((T, BLK), jnp.int32),
            pltpu.VMEM((BLK, EMBED), jnp.float32),
            pltpu.VMEM((BLK, EMBED), jnp.float32),
            pltpu.VMEM((64, 129), jnp.float32),
            pltpu.VMEM((64, 129), jnp.float32),
            pltpu.SemaphoreType.DMA,
            pltpu.SemaphoreType.DMA,
            pltpu.SemaphoreType.DMA,
            pltpu.SemaphoreType.DMA,
        ],
    )
    def emb(xt_hbm, table_hbm, out_hbm, idx_v, rows0, rows1, t0, t1,
            gsem0, gsem1, osem0, osem1):
        wid = lax.axis_index("s") * NC + lax.axis_index("c")
        pltpu.sync_copy(xt_hbm.at[:, wid], idx_v)

        bufs = (rows0, rows1)
        tbufs = (t0, t1)
        gsems = (gsem0, gsem1)
        osems = (osem0, osem1)

        row_ids = [lax.iota(jnp.int32, 16) + 16 * k for k in range(4)]
        zero16 = jnp.zeros((16,), jnp.int32)

        def fire_gather(t, b):
            pltpu.async_copy(
                table_hbm.at[idx_v.at[t]], bufs[b], gsems[b]
            )

        def drain_gather(b):
            pltpu.make_async_copy(
                table_hbm.at[pl.ds(0, BLK)], bufs[b], gsems[b]
            ).wait()

        def fire_out(t, b):
            for e8 in range(8):
                pltpu.async_copy(
                    tbufs[b].at[pl.ds(8 * e8, 8), pl.ds(0, BLK)],
                    out_hbm.at[t, e8, wid],
                    osems[b],
                )

        def drain_out(b):
            for e8 in range(8):
                pltpu.make_async_copy(
                    out_hbm.at[0, e8, 0],
                    tbufs[b].at[pl.ds(8 * e8, 8), pl.ds(0, BLK)],
                    osems[b],
                ).wait()

        def transpose(b):
            @pl.loop(0, BLK, unroll=8)
            def _(br):
                col = zero16 + br
                for k in range(4):
                    v = bufs[b][br, pl.ds(k * 16, 16)]
                    plsc.store_scatter(tbufs[b], [row_ids[k], col], v)

        fire_gather(0, 0)
        fire_gather(1, 1)

        @pl.loop(0, T // 2)
        def _(p):
            for b in range(2):
                t = p * 2 + b
                drain_gather(b)

                @pl.when(t >= 2)
                def _():
                    drain_out(b)

                transpose(b)
                fire_out(t, b)

                @pl.when(t + 2 < T)
                def _():
                    fire_gather(t + 2, b)

        drain_out(0)
        drain_out(1)

    return emb


def kernel(x, table):
    B, T = x.shape
    xt = x.astype(jnp.int32).T.reshape(T, NW, BLK)
    out6 = _build(B, T)(xt, table)
    return out6.transpose(2, 4, 0, 1, 3).reshape(B, T, EMBED)


# speedup vs baseline: 2.5734x; 1.2501x over previous
import functools

import jax
import jax.numpy as jnp
from jax import lax
from jax.experimental import pallas as pl
from jax.experimental.pallas import tpu as pltpu
from jax.experimental.pallas import tpu_sc as plsc

EMBED = 64
NC = 2
NS = 16
NW = NC * NS
BLK = 128


@functools.lru_cache(maxsize=None)
def _build(B, T):
    mesh = plsc.VectorSubcoreMesh(core_axis_name="c", subcore_axis_name="s")

    @functools.partial(
        pl.kernel,
        out_type=jax.ShapeDtypeStruct((T, 8, B // BLK, 8, BLK), jnp.float32),
        mesh=mesh,
        compiler_params=pltpu.CompilerParams(
            use_tc_tiling_on_sc=False, needs_layout_passes=False
        ),
        scratch_types=[
            pltpu.VMEM((T, BLK), jnp.int32),
            pltpu.VMEM((BLK, EMBED), jnp.float32),
            pltpu.VMEM((BLK, EMBED), jnp.float32),
            pltpu.VMEM((64, 129), jnp.float32),
            pltpu.VMEM((64, 129), jnp.float32),
            pltpu.SemaphoreType.DMA,
            pltpu.SemaphoreType.DMA,
            pltpu.SemaphoreType.DMA,
            pltpu.SemaphoreType.DMA,
        ],
    )
    def emb(xt_hbm, table_hbm, out_hbm, idx_v, rows0, rows1, t0, t1,
            gsem0, gsem1, osem0, osem1):
        wid = lax.axis_index("s") * NC + lax.axis_index("c")
        pltpu.sync_copy(xt_hbm.at[:, wid], idx_v)

        bufs = (rows0, rows1)
        tbufs = (t0, t1)
        gsems = (gsem0, gsem1)
        osems = (osem0, osem1)

        row_ids = [lax.iota(jnp.int32, 16) + 16 * k for k in range(4)]
        zero16 = jnp.zeros((16,), jnp.int32)

        def fire_gather(t, b):
            pltpu.async_copy(
                table_hbm.at[idx_v.at[t]], bufs[b], gsems[b]
            )

        def drain_gather(b):
            pltpu.make_async_copy(
                table_hbm.at[pl.ds(0, BLK)], bufs[b], gsems[b]
            ).wait()

        def fire_out(t, b):
            for e8 in range(8):
                pltpu.async_copy(
                    tbufs[b].at[pl.ds(8 * e8, 8), pl.ds(0, BLK)],
                    out_hbm.at[t, e8, wid],
                    osems[b],
                )

        def drain_out(b):
            for e8 in range(8):
                pltpu.make_async_copy(
                    out_hbm.at[0, e8, 0],
                    tbufs[b].at[pl.ds(8 * e8, 8), pl.ds(0, BLK)],
                    osems[b],
                ).wait()

        def transpose(b):
            @plsc.parallel_loop(0, BLK, unroll=8)
            def _(br):
                col = zero16 + br
                for k in range(4):
                    v = bufs[b][br, pl.ds(k * 16, 16)]
                    plsc.store_scatter(tbufs[b], [row_ids[k], col], v)

        fire_gather(0, 0)
        fire_gather(1, 1)

        @pl.loop(0, T // 2)
        def _(p):
            for b in range(2):
                t = p * 2 + b
                drain_gather(b)

                @pl.when(t >= 2)
                def _():
                    drain_out(b)

                transpose(b)
                fire_out(t, b)

                @pl.when(t + 2 < T)
                def _():
                    fire_gather(t + 2, b)

        drain_out(0)
        drain_out(1)

    return emb


def kernel(x, table):
    B, T = x.shape
    xt = x.astype(jnp.int32).T.reshape(T, NW, BLK)
    out6 = _build(B, T)(xt, table)
    return out6.transpose(2, 4, 0, 1, 3).reshape(B, T, EMBED)
